# baseline (speedup 1.0000x reference)
import functools

import jax
import jax.numpy as jnp
from jax import lax
from jax.experimental import pallas as pl
from jax.experimental.pallas import tpu as pltpu
from jax.experimental.pallas import tpu_sc as plsc

N = 100000
RB = 2048
NBLK = 49
NP = RB * NBLK
PAD_ROWS = NP - N
E = 3200000
C = 512
K = 2
NCORE = 2
NSUB = 16
NW = NCORE * NSUB
CPW = 200
CH_MAIN = E // C
CHUNKS = NW * CPW
PADCH = CHUNKS - CH_MAIN
OUTER = CPW // K
SLICE = NP // NSUB


@functools.cache
def _mesh():
    return plsc.VectorSubcoreMesh(core_axis_name="c", subcore_axis_name="s",
                                  num_cores=NCORE, num_subcores=NSUB)


def _wid():
    return lax.axis_index("c") * NSUB + lax.axis_index("s")


def _sc_deg_body(edges_hbm, pads_hbm, zeros_hbm, deg_out,
                 didx, ones_v, acc, lsem, ssem):
    cid = lax.axis_index("c")
    sid = lax.axis_index("s")
    pltpu.sync_copy(zeros_hbm, acc.at[pl.ds(sid * SLICE, SLICE)])
    for i in range(C // 16):
        ones_v[pl.ds(i * 16, 16)] = jnp.full((16,), 1.0, jnp.float32)
    plsc.subcore_barrier()

    wid = _wid()

    def load_idx(blk, b):
        base = wid * CPW + blk * K

        @pl.when(base < CH_MAIN)
        def _():
            pltpu.async_copy(edges_hbm.at[1, pl.ds(base, K)], didx.at[b],
                             lsem)

        @pl.when(base >= CH_MAIN)
        def _():
            pltpu.async_copy(pads_hbm.at[1, pl.ds(base - CH_MAIN, K)],
                             didx.at[b], lsem)

    def wait_idx(b):
        pltpu.make_async_copy(edges_hbm.at[1, pl.ds(0, K)], didx.at[b],
                              lsem).wait()

    def drain_scatters(b):
        for j in range(K):
            pltpu.make_async_copy(ones_v, acc.at[didx.at[b, j]],
                                  ssem).wait()

    load_idx(0, 0)

    def outer(i2, carry):
        for b in range(2):
            blk = i2 * 2 + b

            @pl.when(blk >= 1)
            def _():
                drain_scatters(1 - b)

            @pl.when(blk + 1 < OUTER)
            def _():
                load_idx(blk + 1, 1 - b)

            wait_idx(b)
            for j in range(K):
                pltpu.async_copy(ones_v, acc.at[didx.at[b, j]], ssem,
                                 add=True)
        return carry

    lax.fori_loop(0, OUTER // 2, outer, 0)
    drain_scatters((OUTER - 1) % 2)
    plsc.subcore_barrier()
    pltpu.sync_copy(acc.at[pl.ds(sid * SLICE, SLICE)],
                    deg_out.at[cid, pl.ds(sid * SLICE, SLICE)])


@functools.cache
def _sc_deg():
    return pl.kernel(
        _sc_deg_body,
        out_type=jax.ShapeDtypeStruct((NCORE, NP), jnp.float32),
        mesh=_mesh(),
        scratch_types=[
            pltpu.VMEM((2, K, C), jnp.int32),
            pltpu.VMEM((C,), jnp.float32),
            pltpu.VMEM_SHARED((NP,), jnp.float32),
            pltpu.SemaphoreType.DMA,
            pltpu.SemaphoreType.DMA,
        ],
        compiler_params=pltpu.CompilerParams(use_tc_tiling_on_sc=False),
    )


def _make_sc_prop(w):
    def body(edges_hbm, pads_hbm, tab_hbm, zeros_hbm, s_out,
             eidx, rows, acc, lsem, gsem, ssem):
        cid = lax.axis_index("c")
        sid = lax.axis_index("s")
        pltpu.sync_copy(zeros_hbm, acc.at[pl.ds(sid * SLICE, SLICE), :])
        plsc.subcore_barrier()

        wid = _wid()

        def load_idx(blk, b):
            base = wid * CPW + blk * K

            @pl.when(base < CH_MAIN)
            def _():
                pltpu.async_copy(edges_hbm.at[:, pl.ds(base, K)],
                                 eidx.at[b], lsem)

            @pl.when(base >= CH_MAIN)
            def _():
                pltpu.async_copy(pads_hbm.at[:, pl.ds(base - CH_MAIN, K)],
                                 eidx.at[b], lsem)

        def wait_idx(b):
            pltpu.make_async_copy(edges_hbm.at[:, pl.ds(0, K)], eidx.at[b],
                                  lsem).wait()

        def drain_scatters(b):
            for j in range(K):
                pltpu.make_async_copy(rows.at[b, j],
                                      acc.at[eidx.at[b, 1, j]], ssem).wait()

        load_idx(0, 0)

        def outer(i2, carry):
            for b in range(2):
                blk = i2 * 2 + b

                @pl.when(blk >= 1)
                def _():
                    drain_scatters(1 - b)

                @pl.when(blk + 1 < OUTER)
                def _():
                    load_idx(blk + 1, 1 - b)

                wait_idx(b)
                for j in range(K):
                    pltpu.async_copy(tab_hbm.at[eidx.at[b, 0, j]],
                                     rows.at[b, j], gsem)
                for j in range(K):
                    pltpu.make_async_copy(tab_hbm.at[eidx.at[b, 0, j]],
                                          rows.at[b, j], gsem).wait()
                    pltpu.async_copy(rows.at[b, j], acc.at[eidx.at[b, 1, j]],
                                     ssem, add=True)
            return carry

        lax.fori_loop(0, OUTER // 2, outer, 0)
        drain_scatters((OUTER - 1) % 2)
        plsc.subcore_barrier()
        pltpu.sync_copy(acc.at[pl.ds(sid * SLICE, SLICE), :],
                        s_out.at[cid, pl.ds(sid * SLICE, SLICE), :])

    return pl.kernel(
        body,
        out_type=jax.ShapeDtypeStruct((NCORE, NP, w), jnp.float32),
        mesh=_mesh(),
        scratch_types=[
            pltpu.VMEM((2, 2, K, C), jnp.int32),
            pltpu.VMEM((2, K, C, w), jnp.float32),
            pltpu.VMEM_SHARED((NP, w), jnp.float32),
            pltpu.SemaphoreType.DMA,
            pltpu.SemaphoreType.DMA,
            pltpu.SemaphoreType.DMA,
        ],
        compiler_params=pltpu.CompilerParams(use_tc_tiling_on_sc=False),
    )


_make_sc_prop = functools.cache(_make_sc_prop)


def _prep_body(deg_ref, x_ref, dinv_ref, xp_ref):
    deg = deg_ref[0] + deg_ref[1] + 1.0
    dinv = lax.rsqrt(deg)
    dinv_ref[...] = dinv
    xp_ref[:, :4] = x_ref[...] * dinv
    xp_ref[:, 4:] = jnp.zeros((RB, 4), jnp.float32)


def _tc_prep(deg2, x_pad):
    return pl.pallas_call(
        _prep_body,
        grid=(NBLK,),
        in_specs=[
            pl.BlockSpec((NCORE, RB, 1), lambda i: (0, i, 0)),
            pl.BlockSpec((RB, 4), lambda i: (i, 0)),
        ],
        out_specs=[
            pl.BlockSpec((RB, 1), lambda i: (i, 0)),
            pl.BlockSpec((RB, 8), lambda i: (i, 0)),
        ],
        out_shape=[
            jax.ShapeDtypeStruct((NP, 1), jnp.float32),
            jax.ShapeDtypeStruct((NP, 8), jnp.float32),
        ],
    )(deg2, x_pad)


def _dense1_body(s1_ref, x_ref, dinv_ref, w1_ref, b1_ref, w2_ref,
                 g_ref, gp_ref):
    dinv = dinv_ref[...]
    p1 = (dinv * (s1_ref[0][:, :4] + s1_ref[1][:, :4])
          + (dinv * dinv) * x_ref[...])
    h1 = jnp.maximum(
        jnp.dot(p1, w1_ref[...], preferred_element_type=jnp.float32)
        + b1_ref[...], 0.0)
    g = jnp.dot(h1, w2_ref[...], preferred_element_type=jnp.float32)
    g_ref[...] = g
    gp_ref[...] = dinv * g


def _tc_dense1(s1, x_pad, dinv, W1, b1, W2):
    return pl.pallas_call(
        _dense1_body,
        grid=(NBLK,),
        in_specs=[
            pl.BlockSpec((NCORE, RB, 8), lambda i: (0, i, 0)),
            pl.BlockSpec((RB, 4), lambda i: (i, 0)),
            pl.BlockSpec((RB, 1), lambda i: (i, 0)),
            pl.BlockSpec((4, 16), lambda i: (0, 0)),
            pl.BlockSpec((16,), lambda i: (0,)),
            pl.BlockSpec((16, 8), lambda i: (0, 0)),
        ],
        out_specs=[
            pl.BlockSpec((RB, 8), lambda i: (i, 0)),
            pl.BlockSpec((RB, 8), lambda i: (i, 0)),
        ],
        out_shape=[
            jax.ShapeDtypeStruct((NP, 8), jnp.float32),
            jax.ShapeDtypeStruct((NP, 8), jnp.float32),
        ],
    )(s1, x_pad, dinv, W1, b1, W2)


def _final_body(s2_ref, g_ref, dinv_ref, b2_ref, wfc_ref, bfc_ref, o_ref):
    dinv = dinv_ref[...]
    p2 = dinv * (s2_ref[0] + s2_ref[1]) + (dinv * dinv) * g_ref[...]
    h2 = jnp.maximum(p2 + b2_ref[...], 0.0)
    o_ref[...] = (
        jnp.dot(h2, wfc_ref[...], preferred_element_type=jnp.float32)
        + bfc_ref[...])


def _tc_final(s2, g, dinv, b2, Wfc, bfc):
    return pl.pallas_call(
        _final_body,
        grid=(NBLK,),
        in_specs=[
            pl.BlockSpec((NCORE, RB, 8), lambda i: (0, i, 0)),
            pl.BlockSpec((RB, 8), lambda i: (i, 0)),
            pl.BlockSpec((RB, 1), lambda i: (i, 0)),
            pl.BlockSpec((8,), lambda i: (0,)),
            pl.BlockSpec((8, 1), lambda i: (0, 0)),
            pl.BlockSpec((1,), lambda i: (0,)),
        ],
        out_specs=pl.BlockSpec((RB, 1), lambda i: (i, 0)),
        out_shape=jax.ShapeDtypeStruct((NP, 1), jnp.float32),
    )(s2, g, dinv, b2, Wfc, bfc)


def kernel(x, edge_index, W1, b1, W2, b2, Wfc, bfc):
    x = x.astype(jnp.float32)
    edges = edge_index.astype(jnp.int32).reshape(2, CH_MAIN, C)
    padrow = (N + jnp.arange(PADCH * C, dtype=jnp.int32) % PAD_ROWS
              ).reshape(PADCH, C)
    pads = jnp.stack([padrow, padrow])
    x_pad = jnp.zeros((NP, 4), jnp.float32).at[:N].set(x)
    zeros_d = jnp.zeros((SLICE,), jnp.float32)
    zeros_8 = jnp.zeros((SLICE, 8), jnp.float32)

    deg2 = _sc_deg()(edges, pads, zeros_d)
    dinv, xp = _tc_prep(deg2.reshape(NCORE, NP, 1), x_pad)
    s1 = _make_sc_prop(8)(edges, pads, xp, zeros_8)
    g, gp = _tc_dense1(s1, x_pad, dinv, W1, b1, W2)
    s2 = _make_sc_prop(8)(edges, pads, gp, zeros_8)
    o = _tc_final(s2, g, dinv, b2, Wfc, bfc)
    return o[:N, 0]


# speedup vs baseline: 1.0212x; 1.0212x over previous
import functools

import jax
import jax.numpy as jnp
from jax import lax
from jax.experimental import pallas as pl
from jax.experimental.pallas import tpu as pltpu
from jax.experimental.pallas import tpu_sc as plsc

N = 100000
RB = 3584
NBLK = 28
NP = RB * NBLK
PAD_ROWS = NP - N
E = 3200000
C = 256
K = 4
NCORE = 2
NSUB = 16
NW = NCORE * NSUB
CPW = 400
CH_MAIN = E // C
CHUNKS = NW * CPW
PADCH = CHUNKS - CH_MAIN
OUTER = CPW // K
SLICE = NP // NSUB


@functools.cache
def _mesh():
    return plsc.VectorSubcoreMesh(core_axis_name="c", subcore_axis_name="s",
                                  num_cores=NCORE, num_subcores=NSUB)


def _wid():
    return lax.axis_index("c") * NSUB + lax.axis_index("s")


def _sc_deg_body(edges_hbm, pads_hbm, zeros_hbm, deg_out,
                 didx, ones_v, acc, lsem, ssem):
    cid = lax.axis_index("c")
    sid = lax.axis_index("s")
    pltpu.sync_copy(zeros_hbm, acc.at[pl.ds(sid * SLICE, SLICE)])
    for i in range(C // 16):
        ones_v[pl.ds(i * 16, 16)] = jnp.full((16,), 1.0, jnp.float32)
    plsc.subcore_barrier()

    wid = _wid()

    def load_idx(blk, b):
        base = wid * CPW + blk * K

        @pl.when(base < CH_MAIN)
        def _():
            pltpu.async_copy(edges_hbm.at[1, pl.ds(base, K)], didx.at[b],
                             lsem)

        @pl.when(base >= CH_MAIN)
        def _():
            pltpu.async_copy(pads_hbm.at[1, pl.ds(base - CH_MAIN, K)],
                             didx.at[b], lsem)

    def wait_idx(b):
        pltpu.make_async_copy(edges_hbm.at[1, pl.ds(0, K)], didx.at[b],
                              lsem).wait()

    def drain_scatters(b):
        for j in range(K):
            pltpu.make_async_copy(ones_v, acc.at[didx.at[b, j]],
                                  ssem).wait()

    load_idx(0, 0)

    def outer(i2, carry):
        for b in range(2):
            blk = i2 * 2 + b

            @pl.when(blk >= 1)
            def _():
                drain_scatters(1 - b)

            @pl.when(blk + 1 < OUTER)
            def _():
                load_idx(blk + 1, 1 - b)

            wait_idx(b)
            for j in range(K):
                pltpu.async_copy(ones_v, acc.at[didx.at[b, j]], ssem,
                                 add=True)
        return carry

    lax.fori_loop(0, OUTER // 2, outer, 0)
    drain_scatters((OUTER - 1) % 2)
    plsc.subcore_barrier()
    pltpu.sync_copy(acc.at[pl.ds(sid * SLICE, SLICE)],
                    deg_out.at[cid, pl.ds(sid * SLICE, SLICE)])


@functools.cache
def _sc_deg():
    return pl.kernel(
        _sc_deg_body,
        out_type=jax.ShapeDtypeStruct((NCORE, NP), jnp.float32),
        mesh=_mesh(),
        scratch_types=[
            pltpu.VMEM((2, K, C), jnp.int32),
            pltpu.VMEM((C,), jnp.float32),
            pltpu.VMEM_SHARED((NP,), jnp.float32),
            pltpu.SemaphoreType.DMA,
            pltpu.SemaphoreType.DMA,
        ],
        compiler_params=pltpu.CompilerParams(use_tc_tiling_on_sc=False),
    )


def _make_sc_prop(w):
    def body(edges_hbm, pads_hbm, tab_hbm, zeros_hbm, s_out,
             eidx, rows, acc, lsem, gsem, ssem):
        cid = lax.axis_index("c")
        sid = lax.axis_index("s")
        pltpu.sync_copy(zeros_hbm, acc.at[pl.ds(sid * SLICE, SLICE), :])
        plsc.subcore_barrier()

        wid = _wid()

        def load_idx(blk, b):
            base = wid * CPW + blk * K

            @pl.when(base < CH_MAIN)
            def _():
                pltpu.async_copy(edges_hbm.at[:, pl.ds(base, K)],
                                 eidx.at[b], lsem)

            @pl.when(base >= CH_MAIN)
            def _():
                pltpu.async_copy(pads_hbm.at[:, pl.ds(base - CH_MAIN, K)],
                                 eidx.at[b], lsem)

        def wait_idx(b):
            pltpu.make_async_copy(edges_hbm.at[:, pl.ds(0, K)], eidx.at[b],
                                  lsem).wait()

        def drain_scatters(b):
            for j in range(K):
                pltpu.make_async_copy(rows.at[b, j],
                                      acc.at[eidx.at[b, 1, j]], ssem).wait()

        load_idx(0, 0)

        def outer(i2, carry):
            for b in range(2):
                blk = i2 * 2 + b

                @pl.when(blk >= 1)
                def _():
                    drain_scatters(1 - b)

                @pl.when(blk + 1 < OUTER)
                def _():
                    load_idx(blk + 1, 1 - b)

                wait_idx(b)
                for j in range(K):
                    pltpu.async_copy(tab_hbm.at[eidx.at[b, 0, j]],
                                     rows.at[b, j], gsem)
                for j in range(K):
                    pltpu.make_async_copy(tab_hbm.at[eidx.at[b, 0, j]],
                                          rows.at[b, j], gsem).wait()
                    pltpu.async_copy(rows.at[b, j], acc.at[eidx.at[b, 1, j]],
                                     ssem, add=True)
            return carry

        lax.fori_loop(0, OUTER // 2, outer, 0)
        drain_scatters((OUTER - 1) % 2)
        plsc.subcore_barrier()
        pltpu.sync_copy(acc.at[pl.ds(sid * SLICE, SLICE), :],
                        s_out.at[cid, pl.ds(sid * SLICE, SLICE), :])

    return pl.kernel(
        body,
        out_type=jax.ShapeDtypeStruct((NCORE, NP, w), jnp.float32),
        mesh=_mesh(),
        scratch_types=[
            pltpu.VMEM((2, 2, K, C), jnp.int32),
            pltpu.VMEM((2, K, C, w), jnp.float32),
            pltpu.VMEM_SHARED((NP, w), jnp.float32),
            pltpu.SemaphoreType.DMA,
            pltpu.SemaphoreType.DMA,
            pltpu.SemaphoreType.DMA,
        ],
        compiler_params=pltpu.CompilerParams(use_tc_tiling_on_sc=False),
    )


_make_sc_prop = functools.cache(_make_sc_prop)


def _prep_body(deg_ref, x_ref, dinv_ref, xp_ref):
    deg = deg_ref[0] + deg_ref[1] + 1.0
    dinv = lax.rsqrt(deg)
    dinv_ref[...] = dinv
    xp_ref[:, :4] = x_ref[...] * dinv
    xp_ref[:, 4:] = jnp.zeros((RB, 4), jnp.float32)


def _tc_prep(deg2, x_pad):
    return pl.pallas_call(
        _prep_body,
        grid=(NBLK,),
        in_specs=[
            pl.BlockSpec((NCORE, RB, 1), lambda i: (0, i, 0)),
            pl.BlockSpec((RB, 4), lambda i: (i, 0)),
        ],
        out_specs=[
            pl.BlockSpec((RB, 1), lambda i: (i, 0)),
            pl.BlockSpec((RB, 8), lambda i: (i, 0)),
        ],
        out_shape=[
            jax.ShapeDtypeStruct((NP, 1), jnp.float32),
            jax.ShapeDtypeStruct((NP, 8), jnp.float32),
        ],
    )(deg2, x_pad)


def _dense1_body(s1_ref, x_ref, dinv_ref, w1_ref, b1_ref, w2_ref,
                 g_ref, gp_ref):
    dinv = dinv_ref[...]
    p1 = (dinv * (s1_ref[0][:, :4] + s1_ref[1][:, :4])
          + (dinv * dinv) * x_ref[...])
    h1 = jnp.maximum(
        jnp.dot(p1, w1_ref[...], preferred_element_type=jnp.float32)
        + b1_ref[...], 0.0)
    g = jnp.dot(h1, w2_ref[...], preferred_element_type=jnp.float32)
    g_ref[...] = g
    gp_ref[...] = dinv * g


def _tc_dense1(s1, x_pad, dinv, W1, b1, W2):
    return pl.pallas_call(
        _dense1_body,
        grid=(NBLK,),
        in_specs=[
            pl.BlockSpec((NCORE, RB, 8), lambda i: (0, i, 0)),
            pl.BlockSpec((RB, 4), lambda i: (i, 0)),
            pl.BlockSpec((RB, 1), lambda i: (i, 0)),
            pl.BlockSpec((4, 16), lambda i: (0, 0)),
            pl.BlockSpec((16,), lambda i: (0,)),
            pl.BlockSpec((16, 8), lambda i: (0, 0)),
        ],
        out_specs=[
            pl.BlockSpec((RB, 8), lambda i: (i, 0)),
            pl.BlockSpec((RB, 8), lambda i: (i, 0)),
        ],
        out_shape=[
            jax.ShapeDtypeStruct((NP, 8), jnp.float32),
            jax.ShapeDtypeStruct((NP, 8), jnp.float32),
        ],
    )(s1, x_pad, dinv, W1, b1, W2)


def _final_body(s2_ref, g_ref, dinv_ref, b2_ref, wfc_ref, bfc_ref, o_ref):
    dinv = dinv_ref[...]
    p2 = dinv * (s2_ref[0] + s2_ref[1]) + (dinv * dinv) * g_ref[...]
    h2 = jnp.maximum(p2 + b2_ref[...], 0.0)
    o_ref[...] = (
        jnp.dot(h2, wfc_ref[...], preferred_element_type=jnp.float32)
        + bfc_ref[...])


def _tc_final(s2, g, dinv, b2, Wfc, bfc):
    return pl.pallas_call(
        _final_body,
        grid=(NBLK,),
        in_specs=[
            pl.BlockSpec((NCORE, RB, 8), lambda i: (0, i, 0)),
            pl.BlockSpec((RB, 8), lambda i: (i, 0)),
            pl.BlockSpec((RB, 1), lambda i: (i, 0)),
            pl.BlockSpec((8,), lambda i: (0,)),
            pl.BlockSpec((8, 1), lambda i: (0, 0)),
            pl.BlockSpec((1,), lambda i: (0,)),
        ],
        out_specs=pl.BlockSpec((RB, 1), lambda i: (i, 0)),
        out_shape=jax.ShapeDtypeStruct((NP, 1), jnp.float32),
    )(s2, g, dinv, b2, Wfc, bfc)


def kernel(x, edge_index, W1, b1, W2, b2, Wfc, bfc):
    x = x.astype(jnp.float32)
    edges = edge_index.astype(jnp.int32).reshape(2, CH_MAIN, C)
    padrow = (N + jnp.arange(PADCH * C, dtype=jnp.int32) % PAD_ROWS
              ).reshape(PADCH, C)
    pads = jnp.stack([padrow, padrow])
    x_pad = jnp.zeros((NP, 4), jnp.float32).at[:N].set(x)
    zeros_d = jnp.zeros((SLICE,), jnp.float32)
    zeros_8 = jnp.zeros((SLICE, 8), jnp.float32)

    deg2 = _sc_deg()(edges, pads, zeros_d)
    dinv, xp = _tc_prep(deg2.reshape(NCORE, NP, 1), x_pad)
    s1 = _make_sc_prop(8)(edges, pads, xp, zeros_8)
    g, gp = _tc_dense1(s1, x_pad, dinv, W1, b1, W2)
    s2 = _make_sc_prop(8)(edges, pads, gp, zeros_8)
    o = _tc_final(s2, g, dinv, b2, Wfc, bfc)
    return o[:N, 0]


# speedup vs baseline: 1.0275x; 1.0062x over previous
import functools

import jax
import jax.numpy as jnp
from jax import lax
from jax.experimental import pallas as pl
from jax.experimental.pallas import tpu as pltpu
from jax.experimental.pallas import tpu_sc as plsc

N = 100000
RB = 3584
NBLK = 28
NP = RB * NBLK
PAD_ROWS = NP - N
E = 3200000
C = 128
K = 8
NCORE = 2
NSUB = 16
NW = NCORE * NSUB
CPW = 800
CH_MAIN = E // C
CHUNKS = NW * CPW
PADCH = CHUNKS - CH_MAIN
OUTER = CPW // K
SLICE = NP // NSUB


@functools.cache
def _mesh():
    return plsc.VectorSubcoreMesh(core_axis_name="c", subcore_axis_name="s",
                                  num_cores=NCORE, num_subcores=NSUB)


def _wid():
    return lax.axis_index("c") * NSUB + lax.axis_index("s")


def _sc_deg_body(edges_hbm, pads_hbm, zeros_hbm, deg_out,
                 didx, ones_v, acc, lsem, ssem):
    cid = lax.axis_index("c")
    sid = lax.axis_index("s")
    pltpu.sync_copy(zeros_hbm, acc.at[pl.ds(sid * SLICE, SLICE)])
    for i in range(C // 16):
        ones_v[pl.ds(i * 16, 16)] = jnp.full((16,), 1.0, jnp.float32)
    plsc.subcore_barrier()

    wid = _wid()

    def load_idx(blk, b):
        base = wid * CPW + blk * K

        @pl.when(base < CH_MAIN)
        def _():
            pltpu.async_copy(edges_hbm.at[1, pl.ds(base, K)], didx.at[b],
                             lsem)

        @pl.when(base >= CH_MAIN)
        def _():
            pltpu.async_copy(pads_hbm.at[1, pl.ds(base - CH_MAIN, K)],
                             didx.at[b], lsem)

    def wait_idx(b):
        pltpu.make_async_copy(edges_hbm.at[1, pl.ds(0, K)], didx.at[b],
                              lsem).wait()

    def drain_scatters(b):
        for j in range(K):
            pltpu.make_async_copy(ones_v, acc.at[didx.at[b, j]],
                                  ssem).wait()

    load_idx(0, 0)

    def outer(i2, carry):
        for b in range(2):
            blk = i2 * 2 + b

            @pl.when(blk >= 1)
            def _():
                drain_scatters(1 - b)

            @pl.when(blk + 1 < OUTER)
            def _():
                load_idx(blk + 1, 1 - b)

            wait_idx(b)
            for j in range(K):
                pltpu.async_copy(ones_v, acc.at[didx.at[b, j]], ssem,
                                 add=True)
        return carry

    lax.fori_loop(0, OUTER // 2, outer, 0)
    drain_scatters((OUTER - 1) % 2)
    plsc.subcore_barrier()
    pltpu.sync_copy(acc.at[pl.ds(sid * SLICE, SLICE)],
                    deg_out.at[cid, pl.ds(sid * SLICE, SLICE)])


@functools.cache
def _sc_deg():
    return pl.kernel(
        _sc_deg_body,
        out_type=jax.ShapeDtypeStruct((NCORE, NP), jnp.float32),
        mesh=_mesh(),
        scratch_types=[
            pltpu.VMEM((2, K, C), jnp.int32),
            pltpu.VMEM((C,), jnp.float32),
            pltpu.VMEM_SHARED((NP,), jnp.float32),
            pltpu.SemaphoreType.DMA,
            pltpu.SemaphoreType.DMA,
        ],
        compiler_params=pltpu.CompilerParams(use_tc_tiling_on_sc=False),
    )


def _make_sc_prop(w):
    def body(edges_hbm, pads_hbm, tab_hbm, zeros_hbm, s_out,
             eidx, rows, acc, lsem, gsem, ssem):
        cid = lax.axis_index("c")
        sid = lax.axis_index("s")
        pltpu.sync_copy(zeros_hbm, acc.at[pl.ds(sid * SLICE, SLICE), :])
        plsc.subcore_barrier()

        wid = _wid()

        def load_idx(blk, b):
            base = wid * CPW + blk * K

            @pl.when(base < CH_MAIN)
            def _():
                pltpu.async_copy(edges_hbm.at[:, pl.ds(base, K)],
                                 eidx.at[b], lsem)

            @pl.when(base >= CH_MAIN)
            def _():
                pltpu.async_copy(pads_hbm.at[:, pl.ds(base - CH_MAIN, K)],
                                 eidx.at[b], lsem)

        def wait_idx(b):
            pltpu.make_async_copy(edges_hbm.at[:, pl.ds(0, K)], eidx.at[b],
                                  lsem).wait()

        def drain_scatters(b):
            for j in range(K):
                pltpu.make_async_copy(rows.at[b, j],
                                      acc.at[eidx.at[b, 1, j]], ssem).wait()

        load_idx(0, 0)

        def outer(i2, carry):
            for b in range(2):
                blk = i2 * 2 + b

                @pl.when(blk >= 1)
                def _():
                    drain_scatters(1 - b)

                @pl.when(blk + 1 < OUTER)
                def _():
                    load_idx(blk + 1, 1 - b)

                wait_idx(b)
                for j in range(K):
                    pltpu.async_copy(tab_hbm.at[eidx.at[b, 0, j]],
                                     rows.at[b, j], gsem)
                for j in range(K):
                    pltpu.make_async_copy(tab_hbm.at[eidx.at[b, 0, j]],
                                          rows.at[b, j], gsem).wait()
                    pltpu.async_copy(rows.at[b, j], acc.at[eidx.at[b, 1, j]],
                                     ssem, add=True)
            return carry

        lax.fori_loop(0, OUTER // 2, outer, 0)
        drain_scatters((OUTER - 1) % 2)
        plsc.subcore_barrier()
        pltpu.sync_copy(acc.at[pl.ds(sid * SLICE, SLICE), :],
                        s_out.at[cid, pl.ds(sid * SLICE, SLICE), :])

    return pl.kernel(
        body,
        out_type=jax.ShapeDtypeStruct((NCORE, NP, w), jnp.float32),
        mesh=_mesh(),
        scratch_types=[
            pltpu.VMEM((2, 2, K, C), jnp.int32),
            pltpu.VMEM((2, K, C, w), jnp.float32),
            pltpu.VMEM_SHARED((NP, w), jnp.float32),
            pltpu.SemaphoreType.DMA,
            pltpu.SemaphoreType.DMA,
            pltpu.SemaphoreType.DMA,
        ],
        compiler_params=pltpu.CompilerParams(use_tc_tiling_on_sc=False),
    )


_make_sc_prop = functools.cache(_make_sc_prop)


def _prep_body(deg_ref, x_ref, dinv_ref, xp_ref):
    deg = deg_ref[0] + deg_ref[1] + 1.0
    dinv = lax.rsqrt(deg)
    dinv_ref[...] = dinv
    xp_ref[:, :4] = x_ref[...] * dinv
    xp_ref[:, 4:] = jnp.zeros((RB, 4), jnp.float32)


def _tc_prep(deg2, x_pad):
    return pl.pallas_call(
        _prep_body,
        grid=(NBLK,),
        in_specs=[
            pl.BlockSpec((NCORE, RB, 1), lambda i: (0, i, 0)),
            pl.BlockSpec((RB, 4), lambda i: (i, 0)),
        ],
        out_specs=[
            pl.BlockSpec((RB, 1), lambda i: (i, 0)),
            pl.BlockSpec((RB, 8), lambda i: (i, 0)),
        ],
        out_shape=[
            jax.ShapeDtypeStruct((NP, 1), jnp.float32),
            jax.ShapeDtypeStruct((NP, 8), jnp.float32),
        ],
    )(deg2, x_pad)


def _dense1_body(s1_ref, x_ref, dinv_ref, w1_ref, b1_ref, w2_ref,
                 g_ref, gp_ref):
    dinv = dinv_ref[...]
    p1 = (dinv * (s1_ref[0][:, :4] + s1_ref[1][:, :4])
          + (dinv * dinv) * x_ref[...])
    h1 = jnp.maximum(
        jnp.dot(p1, w1_ref[...], preferred_element_type=jnp.float32)
        + b1_ref[...], 0.0)
    g = jnp.dot(h1, w2_ref[...], preferred_element_type=jnp.float32)
    g_ref[...] = g
    gp_ref[...] = dinv * g


def _tc_dense1(s1, x_pad, dinv, W1, b1, W2):
    return pl.pallas_call(
        _dense1_body,
        grid=(NBLK,),
        in_specs=[
            pl.BlockSpec((NCORE, RB, 8), lambda i: (0, i, 0)),
            pl.BlockSpec((RB, 4), lambda i: (i, 0)),
            pl.BlockSpec((RB, 1), lambda i: (i, 0)),
            pl.BlockSpec((4, 16), lambda i: (0, 0)),
            pl.BlockSpec((16,), lambda i: (0,)),
            pl.BlockSpec((16, 8), lambda i: (0, 0)),
        ],
        out_specs=[
            pl.BlockSpec((RB, 8), lambda i: (i, 0)),
            pl.BlockSpec((RB, 8), lambda i: (i, 0)),
        ],
        out_shape=[
            jax.ShapeDtypeStruct((NP, 8), jnp.float32),
            jax.ShapeDtypeStruct((NP, 8), jnp.float32),
        ],
    )(s1, x_pad, dinv, W1, b1, W2)


def _final_body(s2_ref, g_ref, dinv_ref, b2_ref, wfc_ref, bfc_ref, o_ref):
    dinv = dinv_ref[...]
    p2 = dinv * (s2_ref[0] + s2_ref[1]) + (dinv * dinv) * g_ref[...]
    h2 = jnp.maximum(p2 + b2_ref[...], 0.0)
    o_ref[...] = (
        jnp.dot(h2, wfc_ref[...], preferred_element_type=jnp.float32)
        + bfc_ref[...])


def _tc_final(s2, g, dinv, b2, Wfc, bfc):
    return pl.pallas_call(
        _final_body,
        grid=(NBLK,),
        in_specs=[
            pl.BlockSpec((NCORE, RB, 8), lambda i: (0, i, 0)),
            pl.BlockSpec((RB, 8), lambda i: (i, 0)),
            pl.BlockSpec((RB, 1), lambda i: (i, 0)),
            pl.BlockSpec((8,), lambda i: (0,)),
            pl.BlockSpec((8, 1), lambda i: (0, 0)),
            pl.BlockSpec((1,), lambda i: (0,)),
        ],
        out_specs=pl.BlockSpec((RB, 1), lambda i: (i, 0)),
        out_shape=jax.ShapeDtypeStruct((NP, 1), jnp.float32),
    )(s2, g, dinv, b2, Wfc, bfc)


def kernel(x, edge_index, W1, b1, W2, b2, Wfc, bfc):
    x = x.astype(jnp.float32)
    edges = edge_index.astype(jnp.int32).reshape(2, CH_MAIN, C)
    padrow = (N + jnp.arange(PADCH * C, dtype=jnp.int32) % PAD_ROWS
              ).reshape(PADCH, C)
    pads = jnp.stack([padrow, padrow])
    x_pad = jnp.zeros((NP, 4), jnp.float32).at[:N].set(x)
    zeros_d = jnp.zeros((SLICE,), jnp.float32)
    zeros_8 = jnp.zeros((SLICE, 8), jnp.float32)

    deg2 = _sc_deg()(edges, pads, zeros_d)
    dinv, xp = _tc_prep(deg2.reshape(NCORE, NP, 1), x_pad)
    s1 = _make_sc_prop(8)(edges, pads, xp, zeros_8)
    g, gp = _tc_dense1(s1, x_pad, dinv, W1, b1, W2)
    s2 = _make_sc_prop(8)(edges, pads, gp, zeros_8)
    o = _tc_final(s2, g, dinv, b2, Wfc, bfc)
    return o[:N, 0]


# speedup vs baseline: 1.2442x; 1.2109x over previous
import functools

import jax
import jax.numpy as jnp
from jax import lax
from jax.experimental import pallas as pl
from jax.experimental.pallas import tpu as pltpu
from jax.experimental.pallas import tpu_sc as plsc

N = 100000
RB = 3584
NBLK = 28
NP = RB * NBLK
PAD_ROWS = NP - N
E = 3200000
C = 128
K = 16
NCORE = 2
NSUB = 16
NW = NCORE * NSUB
CPW = 800
CH_MAIN = E // C
CH_CUT = (CH_MAIN // K) * K
PADCH = NW * CPW - CH_CUT
CHUNKS = NW * CPW
OUTER = CPW // K
SLICE = NP // NSUB


@functools.cache
def _mesh():
    return plsc.VectorSubcoreMesh(core_axis_name="c", subcore_axis_name="s",
                                  num_cores=NCORE, num_subcores=NSUB)


def _wid():
    return lax.axis_index("c") * NSUB + lax.axis_index("s")


def _sc_deg_body(edges_hbm, pads_hbm, zeros_hbm, deg_out,
                 didx, ones_v, acc, lsem, ssem):
    cid = lax.axis_index("c")
    sid = lax.axis_index("s")
    pltpu.sync_copy(zeros_hbm, acc.at[pl.ds(sid * SLICE, SLICE)])
    for i in range(C // 16):
        ones_v[pl.ds(i * 16, 16)] = jnp.full((16,), 1.0, jnp.float32)
    plsc.subcore_barrier()

    wid = _wid()

    def load_idx(blk, b):
        base = wid * CPW + blk * K

        @pl.when(base < CH_CUT)
        def _():
            pltpu.async_copy(edges_hbm.at[1, pl.ds(base, K)], didx.at[b],
                             lsem)

        @pl.when(base >= CH_CUT)
        def _():
            pltpu.async_copy(pads_hbm.at[1, pl.ds(base - CH_CUT, K)],
                             didx.at[b], lsem)

    def wait_idx(b):
        pltpu.make_async_copy(edges_hbm.at[1, pl.ds(0, K)], didx.at[b],
                              lsem).wait()

    def drain_scatters(b):
        for j in range(K):
            pltpu.make_async_copy(ones_v, acc.at[didx.at[b, j]],
                                  ssem).wait()

    load_idx(0, 0)

    def outer(i2, carry):
        for b in range(2):
            blk = i2 * 2 + b

            @pl.when(blk >= 1)
            def _():
                drain_scatters(1 - b)

            @pl.when(blk + 1 < OUTER)
            def _():
                load_idx(blk + 1, 1 - b)

            wait_idx(b)
            for j in range(K):
                pltpu.async_copy(ones_v, acc.at[didx.at[b, j]], ssem,
                                 add=True)
        return carry

    lax.fori_loop(0, OUTER // 2, outer, 0)
    drain_scatters((OUTER - 1) % 2)
    plsc.subcore_barrier()
    pltpu.sync_copy(acc.at[pl.ds(sid * SLICE, SLICE)],
                    deg_out.at[cid, pl.ds(sid * SLICE, SLICE)])


@functools.cache
def _sc_deg():
    return pl.kernel(
        _sc_deg_body,
        out_type=jax.ShapeDtypeStruct((NCORE, NP), jnp.float32),
        mesh=_mesh(),
        scratch_types=[
            pltpu.VMEM((2, K, C), jnp.int32),
            pltpu.VMEM((C,), jnp.float32),
            pltpu.VMEM_SHARED((NP,), jnp.float32),
            pltpu.SemaphoreType.DMA,
            pltpu.SemaphoreType.DMA,
        ],
        compiler_params=pltpu.CompilerParams(use_tc_tiling_on_sc=False),
    )


def _make_sc_prop(w):
    def body(edges_hbm, pads_hbm, tab_hbm, zeros_hbm, s_out,
             eidx, rows, acc, lsem, gsem, ssem):
        cid = lax.axis_index("c")
        sid = lax.axis_index("s")
        pltpu.sync_copy(zeros_hbm, acc.at[pl.ds(sid * SLICE, SLICE), :])
        plsc.subcore_barrier()

        wid = _wid()

        def load_idx(blk, b):
            base = wid * CPW + blk * K

            @pl.when(base < CH_CUT)
            def _():
                pltpu.async_copy(edges_hbm.at[:, pl.ds(base, K)],
                                 eidx.at[b], lsem)

            @pl.when(base >= CH_CUT)
            def _():
                pltpu.async_copy(pads_hbm.at[:, pl.ds(base - CH_CUT, K)],
                                 eidx.at[b], lsem)

        def wait_idx(b):
            pltpu.make_async_copy(edges_hbm.at[:, pl.ds(0, K)], eidx.at[b],
                                  lsem).wait()

        def drain_scatters(b):
            for j in range(K):
                pltpu.make_async_copy(rows.at[b, j],
                                      acc.at[eidx.at[b, 1, j]], ssem).wait()

        load_idx(0, 0)

        def outer(i2, carry):
            for b in range(2):
                blk = i2 * 2 + b

                @pl.when(blk >= 1)
                def _():
                    drain_scatters(1 - b)

                @pl.when(blk + 1 < OUTER)
                def _():
                    load_idx(blk + 1, 1 - b)

                wait_idx(b)
                for j in range(K):
                    pltpu.async_copy(tab_hbm.at[eidx.at[b, 0, j]],
                                     rows.at[b, j], gsem)
                for j in range(K):
                    pltpu.make_async_copy(tab_hbm.at[eidx.at[b, 0, j]],
                                          rows.at[b, j], gsem).wait()
                    pltpu.async_copy(rows.at[b, j], acc.at[eidx.at[b, 1, j]],
                                     ssem, add=True)
            return carry

        lax.fori_loop(0, OUTER // 2, outer, 0)
        drain_scatters((OUTER - 1) % 2)
        plsc.subcore_barrier()
        pltpu.sync_copy(acc.at[pl.ds(sid * SLICE, SLICE), :],
                        s_out.at[cid, pl.ds(sid * SLICE, SLICE), :])

    return pl.kernel(
        body,
        out_type=jax.ShapeDtypeStruct((NCORE, NP, w), jnp.float32),
        mesh=_mesh(),
        scratch_types=[
            pltpu.VMEM((2, 2, K, C), jnp.int32),
            pltpu.VMEM((2, K, C, w), jnp.float32),
            pltpu.VMEM_SHARED((NP, w), jnp.float32),
            pltpu.SemaphoreType.DMA,
            pltpu.SemaphoreType.DMA,
            pltpu.SemaphoreType.DMA,
        ],
        compiler_params=pltpu.CompilerParams(use_tc_tiling_on_sc=False),
    )


_make_sc_prop = functools.cache(_make_sc_prop)


def _prep_body(deg_ref, x_ref, dinv_ref, xp_ref):
    deg = deg_ref[0, :] + deg_ref[1, :] + 1.0
    dinv = lax.rsqrt(deg)
    dinv_ref[...] = dinv[None, None, :]
    xp_ref[:, :4] = x_ref[...] * dinv[:, None]
    xp_ref[:, 4:] = jnp.zeros((RB, 4), jnp.float32)


def _tc_prep(deg2, x_pad):
    return pl.pallas_call(
        _prep_body,
        grid=(NBLK,),
        in_specs=[
            pl.BlockSpec((NCORE, RB), lambda i: (0, i)),
            pl.BlockSpec((RB, 4), lambda i: (i, 0)),
        ],
        out_specs=[
            pl.BlockSpec((1, 1, RB), lambda i: (i, 0, 0)),
            pl.BlockSpec((RB, 8), lambda i: (i, 0)),
        ],
        out_shape=[
            jax.ShapeDtypeStruct((NBLK, 1, RB), jnp.float32),
            jax.ShapeDtypeStruct((NP, 8), jnp.float32),
        ],
    )(deg2, x_pad)


def _dense1_body(s1_ref, x_ref, dinv_ref, w1_ref, b1_ref, w2_ref,
                 g_ref, gp_ref):
    dinv = dinv_ref[0, 0, :][:, None]
    p1 = (dinv * (s1_ref[0][:, :4] + s1_ref[1][:, :4])
          + (dinv * dinv) * x_ref[...])
    h1 = jnp.maximum(
        jnp.dot(p1, w1_ref[...], preferred_element_type=jnp.float32)
        + b1_ref[...], 0.0)
    g = jnp.dot(h1, w2_ref[...], preferred_element_type=jnp.float32)
    g_ref[...] = g
    gp_ref[...] = dinv * g


def _tc_dense1(s1, x_pad, dinv, W1, b1, W2):
    return pl.pallas_call(
        _dense1_body,
        grid=(NBLK,),
        in_specs=[
            pl.BlockSpec((NCORE, RB, 8), lambda i: (0, i, 0)),
            pl.BlockSpec((RB, 4), lambda i: (i, 0)),
            pl.BlockSpec((1, 1, RB), lambda i: (i, 0, 0)),
            pl.BlockSpec((4, 16), lambda i: (0, 0)),
            pl.BlockSpec((16,), lambda i: (0,)),
            pl.BlockSpec((16, 8), lambda i: (0, 0)),
        ],
        out_specs=[
            pl.BlockSpec((RB, 8), lambda i: (i, 0)),
            pl.BlockSpec((RB, 8), lambda i: (i, 0)),
        ],
        out_shape=[
            jax.ShapeDtypeStruct((NP, 8), jnp.float32),
            jax.ShapeDtypeStruct((NP, 8), jnp.float32),
        ],
    )(s1, x_pad, dinv, W1, b1, W2)


def _final_body(s2_ref, g_ref, dinv_ref, b2_ref, wfc_ref, bfc_ref, o_ref):
    dinv = dinv_ref[0, 0, :][:, None]
    p2 = dinv * (s2_ref[0] + s2_ref[1]) + (dinv * dinv) * g_ref[...]
    h2 = jnp.maximum(p2 + b2_ref[...], 0.0)
    o_ref[...] = (jnp.dot(h2, wfc_ref[...],
                          preferred_element_type=jnp.float32)[:, 0]
                  + bfc_ref[0])[None, None, :]


def _tc_final(s2, g, dinv, b2, Wfc, bfc):
    return pl.pallas_call(
        _final_body,
        grid=(NBLK,),
        in_specs=[
            pl.BlockSpec((NCORE, RB, 8), lambda i: (0, i, 0)),
            pl.BlockSpec((RB, 8), lambda i: (i, 0)),
            pl.BlockSpec((1, 1, RB), lambda i: (i, 0, 0)),
            pl.BlockSpec((8,), lambda i: (0,)),
            pl.BlockSpec((8, 1), lambda i: (0, 0)),
            pl.BlockSpec((1,), lambda i: (0,)),
        ],
        out_specs=pl.BlockSpec((1, 1, RB), lambda i: (i, 0, 0)),
        out_shape=jax.ShapeDtypeStruct((NBLK, 1, RB), jnp.float32),
    )(s2, g, dinv, b2, Wfc, bfc)


def kernel(x, edge_index, W1, b1, W2, b2, Wfc, bfc):
    x = x.astype(jnp.float32)
    edges = edge_index.astype(jnp.int32).reshape(2, CH_MAIN, C)
    nfill = PADCH - (CH_MAIN - CH_CUT)
    fill = (N + jnp.arange(nfill * C, dtype=jnp.int32) % PAD_ROWS
            ).reshape(1, nfill, C)
    pads = jnp.concatenate(
        [edges[:, CH_CUT:], jnp.broadcast_to(fill, (2, nfill, C))], axis=1)
    x_pad = jnp.concatenate(
        [x.reshape(-1), jnp.zeros(4 * (NP - N), jnp.float32)]).reshape(NP, 4)
    zeros_d = jnp.zeros((SLICE,), jnp.float32)
    zeros_8 = jnp.zeros((SLICE, 8), jnp.float32)

    deg2 = _sc_deg()(edges, pads, zeros_d)
    dinv, xp = _tc_prep(deg2, x_pad)
    s1 = _make_sc_prop(8)(edges, pads, xp, zeros_8)
    g, gp = _tc_dense1(s1, x_pad, dinv, W1, b1, W2)
    s2 = _make_sc_prop(8)(edges, pads, gp, zeros_8)
    o = _tc_final(s2, g, dinv, b2, Wfc, bfc)
    return o.reshape(NP)[:N]
